# Initial kernel scaffold; baseline (speedup 1.0000x reference)
import jax
import jax.numpy as jnp
from jax.experimental import pallas as pl


def kernel(sequence, pe):
    raise NotImplementedError("write your pallas kernel here")


# speedup vs baseline: 1.6986x; 1.6986x over previous
import jax
import jax.numpy as jnp
from jax.experimental import pallas as pl

_BS = 512


def _add_kernel(seq_ref, pe_ref, out_ref):
    out_ref[...] = seq_ref[...] + pe_ref[...]


def kernel(sequence, pe):
    b, s, d = sequence.shape
    ns = s // _BS
    return pl.pallas_call(
        _add_kernel,
        grid=(ns, b),
        in_specs=[
            pl.BlockSpec((1, _BS, d), lambda i, j: (j, i, 0)),
            pl.BlockSpec((_BS, d), lambda i, j: (i, 0)),
        ],
        out_specs=pl.BlockSpec((1, _BS, d), lambda i, j: (j, i, 0)),
        out_shape=jax.ShapeDtypeStruct((b, s, d), sequence.dtype),
    )(sequence, pe)


# speedup vs baseline: 1.8810x; 1.1074x over previous
import jax
import jax.numpy as jnp
from jax.experimental import pallas as pl

_BS = 1024


def _add_kernel(seq_ref, pe_ref, out_ref):
    out_ref[...] = seq_ref[...] + pe_ref[...]


def kernel(sequence, pe):
    b, s, d = sequence.shape
    ns = s // _BS
    return pl.pallas_call(
        _add_kernel,
        grid=(ns, b),
        in_specs=[
            pl.BlockSpec((1, _BS, d), lambda i, j: (j, i, 0)),
            pl.BlockSpec((_BS, d), lambda i, j: (i, 0)),
        ],
        out_specs=pl.BlockSpec((1, _BS, d), lambda i, j: (j, i, 0)),
        out_shape=jax.ShapeDtypeStruct((b, s, d), sequence.dtype),
    )(sequence, pe)


# speedup vs baseline: 1.9887x; 1.0572x over previous
import jax
import jax.numpy as jnp
from jax.experimental import pallas as pl

_BS = 2048


def _add_kernel(seq_ref, pe_ref, out_ref):
    out_ref[...] = seq_ref[...] + pe_ref[...]


def kernel(sequence, pe):
    b, s, d = sequence.shape
    ns = s // _BS
    return pl.pallas_call(
        _add_kernel,
        grid=(ns, b),
        in_specs=[
            pl.BlockSpec((1, _BS, d), lambda i, j: (j, i, 0)),
            pl.BlockSpec((_BS, d), lambda i, j: (i, 0)),
        ],
        out_specs=pl.BlockSpec((1, _BS, d), lambda i, j: (j, i, 0)),
        out_shape=jax.ShapeDtypeStruct((b, s, d), sequence.dtype),
    )(sequence, pe)
